# baseline (speedup 1.0000x reference)
import functools

import jax
import jax.numpy as jnp
from jax import lax
from jax.experimental import pallas as pl
from jax.experimental.pallas import tpu as pltpu
from jax.experimental.pallas import tpu_sc as plsc

F = 256
HALF = F // 2
TRIL = F * (F + 1) // 2
S_HALF = HALF * (HALF + 1) // 2
IN_LO = S_HALF
IN_HI = TRIL - S_HALF
OUT = F * F
OUT_HALF = OUT // 2
DIAG_OFFSET = 0.5
NC = 2
NS = 16
NW = NC * NS


def _sc_body(in_hbm, out_hbm, outb, inb, sem_in_hi, sem_in_lo,
             sem_out_hi, sem_out_lo):
    cid = lax.axis_index("c")
    sid = lax.axis_index("s")
    wid = sid * NC + cid
    batch = in_hbm.shape[0] // TRIL
    per_w = batch // NW
    base = wid * per_w

    iota16 = lax.iota(jnp.int32, 16)
    zeros16 = jnp.zeros((16,), jnp.float32)

    def in_hi_copy(b):
        return pltpu.make_async_copy(
            in_hbm.at[pl.ds(b * TRIL + S_HALF, IN_HI)],
            inb.at[pl.ds(S_HALF, IN_HI)], sem_in_hi)

    def in_lo_copy(b):
        return pltpu.make_async_copy(
            in_hbm.at[pl.ds(b * TRIL, IN_LO)], inb.at[pl.ds(0, IN_LO)],
            sem_in_lo)

    def out_hi_copy(b):
        return pltpu.make_async_copy(
            outb.at[pl.ds(OUT_HALF, OUT_HALF)],
            out_hbm.at[pl.ds(b * OUT + OUT_HALF, OUT_HALF)], sem_out_hi)

    def out_lo_copy(b):
        return pltpu.make_async_copy(
            outb.at[pl.ds(0, OUT_HALF)],
            out_hbm.at[pl.ds(b * OUT, OUT_HALF)], sem_out_lo)

    def expand(r0, r1):
        for j in range(F // 16):
            col = j * 16
            lo = max(col + 16, r0)
            if lo >= r1:
                continue

            @plsc.parallel_loop(lo, r1, unroll=8)
            def _copy(r):
                s = (r * (r + 1)) >> 1
                outb[pl.ds(r * F + col, 16)] = inb[pl.ds(s + col, 16)]

        @plsc.parallel_loop(r0, r1, unroll=4)
        def _diag(r):
            s = (r * (r + 1)) >> 1
            jd16 = (r >> 4) * 16
            c = jd16 + iota16
            vals = inb[pl.ds(s + jd16, 16)]
            dval = jnp.abs(DIAG_OFFSET + vals) + 1e-9
            res = jnp.where(c < r, vals, jnp.where(c == r, dval, zeros16))
            outb[pl.ds(r * F + jd16, 16)] = res

    @plsc.parallel_loop(0, OUT // 16, unroll=8)
    def _zero(k):
        outb[pl.ds(k * 16, 16)] = zeros16

    pltpu.sync_copy(in_hbm.at[pl.ds(base * TRIL, TRIL)], inb)

    def batch_body(t, _):
        b = base + t

        @pl.when(t > 0)
        def _():
            in_hi_copy(b).wait()
            out_hi_copy(b - 1).wait()
        expand(HALF, F)
        out_hi_copy(b).start()

        @pl.when(t < per_w - 1)
        def _():
            in_hi_copy(b + 1).start()

        @pl.when(t > 0)
        def _():
            in_lo_copy(b).wait()
            out_lo_copy(b - 1).wait()
        expand(0, HALF)
        out_lo_copy(b).start()

        @pl.when(t < per_w - 1)
        def _():
            in_lo_copy(b + 1).start()
        return 0
    lax.fori_loop(0, per_w, batch_body, 0)

    out_hi_copy(base + per_w - 1).wait()
    out_lo_copy(base + per_w - 1).wait()


def kernel(input):
    batch = input.shape[0]
    mesh = plsc.VectorSubcoreMesh(core_axis_name="c", subcore_axis_name="s")
    run = functools.partial(
        pl.kernel,
        mesh=mesh,
        out_type=jax.ShapeDtypeStruct((batch * OUT,), jnp.float32),
        scratch_types=[
            pltpu.VMEM((OUT,), jnp.float32),
            pltpu.VMEM((TRIL,), jnp.float32),
            pltpu.SemaphoreType.DMA,
            pltpu.SemaphoreType.DMA,
            pltpu.SemaphoreType.DMA,
            pltpu.SemaphoreType.DMA,
        ],
    )(_sc_body)
    flat = run(input.reshape(-1))
    return flat.reshape(batch, F, F)
